# baseline (speedup 1.0000x reference)
import functools

import jax
import jax.numpy as jnp
from jax import lax
from jax.experimental import pallas as pl
from jax.experimental.pallas import tpu as pltpu
from jax.experimental.pallas import tpu_sc as plsc

N = 10000
E = 320000
D = 128

NC = 2
NS = 16
NW = NC * NS

EPW = E // NW
KE = 80
NBE = EPW // KE
PHB = 64
ROWS_PT = 624
ROWS_TAIL = N - NS * ROWS_PT

E2 = 2 * E
SPW = E2 // NW
KS = 80
NBS = SPW // KS

_MESH = plsc.VectorSubcoreMesh(
    core_axis_name="c", subcore_axis_name="s", num_cores=NC, num_subcores=NS)
_SC_PARAMS = pltpu.CompilerParams(needs_layout_passes=False)


def _zero_acc(zrows, acc, sid):
    rbase = sid * ROWS_PT
    tbase = NS * ROWS_PT
    pltpu.sync_copy(zrows.at[pl.ds(rbase, ROWS_PT)], acc.at[pl.ds(rbase, ROWS_PT)])

    @pl.when(sid == 0)
    def _():
        pltpu.sync_copy(zrows.at[pl.ds(tbase, ROWS_TAIL)],
                        acc.at[pl.ds(tbase, ROWS_TAIL)])


def _copy_out(acc, out, cid, sid):
    rbase = sid * ROWS_PT
    tbase = NS * ROWS_PT
    pltpu.sync_copy(acc.at[pl.ds(rbase, ROWS_PT)],
                    out.at[cid].at[pl.ds(rbase, ROWS_PT)])

    @pl.when(sid == 0)
    def _():
        pltpu.sync_copy(acc.at[pl.ds(tbase, ROWS_TAIL)],
                        out.at[cid].at[pl.ds(tbase, ROWS_TAIL)])


def _agg_body(h, src3, dst3, zrows, agg_out, deg_out,
              sidx, didx, rows_a, rows_b, degv, acc, sga, sgb, ssa, ssb):
    cid = lax.axis_index("c")
    sid = lax.axis_index("s")
    wid = cid * NS + sid

    _zero_acc(zrows, acc, sid)

    z16 = jnp.zeros((16,), jnp.float32)

    def zblk(i, carry):
        degv[pl.ds(i * 16, 16)] = z16
        return carry

    lax.fori_loop(0, N // 16, zblk, 0)
    plsc.subcore_barrier()

    def gather(j, rows, sem):
        return pltpu.async_copy(h.at[sidx.at[j]], rows, sem)

    def gather_wait(rows, sem):
        pltpu.make_async_copy(h.at[sidx.at[0]], rows, sem).wait()

    def scat(j, rows, sem):
        return pltpu.async_copy(rows, acc.at[didx.at[j]], sem, add=True)

    def scat_wait(rows, sem):
        pltpu.make_async_copy(rows, acc.at[didx.at[0]], sem).wait()

    ones16 = jnp.ones((16,), jnp.float32)

    def degcount(j):
        for g in range(KE // 16):
            ids = didx[j, pl.ds(g * 16, 16)]
            plsc.addupdate_scatter(degv, [ids], ones16)

    for blk0, pnb in ((0, PHB), (PHB, NBE - PHB)):
        pltpu.sync_copy(src3.at[wid].at[pl.ds(blk0, pnb)],
                        sidx.at[pl.ds(0, pnb)])
        pltpu.sync_copy(dst3.at[wid].at[pl.ds(blk0, pnb)],
                        didx.at[pl.ds(0, pnb)])

        gather(0, rows_a, sga)

        def blk(i, carry):
            a = 2 * i
            b = a + 1
            gather_wait(rows_a, sga)
            gather(b, rows_b, sgb)
            scat(a, rows_a, ssa)
            degcount(a)
            gather_wait(rows_b, sgb)
            scat_wait(rows_a, ssa)
            gather(jnp.minimum(a + 2, pnb - 1), rows_a, sga)
            scat(b, rows_b, ssb)
            degcount(b)
            scat_wait(rows_b, ssb)
            return carry

        lax.fori_loop(0, pnb // 2, blk, 0)
        gather_wait(rows_a, sga)
        if pnb % 2:
            scat(pnb - 1, rows_a, ssa)
            degcount(pnb - 1)
            scat_wait(rows_a, ssa)

    plsc.subcore_barrier()
    _copy_out(acc, agg_out, cid, sid)
    pltpu.sync_copy(degv, deg_out.at[cid].at[sid])


_agg = pl.kernel(
    _agg_body,
    out_type=(jax.ShapeDtypeStruct((NC, N, D), jnp.float32),
              jax.ShapeDtypeStruct((NC, NS, N), jnp.float32)),
    mesh=_MESH,
    scratch_types=[
        pltpu.VMEM((PHB, KE), jnp.int32),
        pltpu.VMEM((PHB, KE), jnp.int32),
        pltpu.VMEM((KE, D), jnp.float32),
        pltpu.VMEM((KE, D), jnp.float32),
        pltpu.VMEM((N,), jnp.float32),
        pltpu.VMEM_SHARED((N, D), jnp.float32),
        pltpu.SemaphoreType.DMA,
        pltpu.SemaphoreType.DMA,
        pltpu.SemaphoreType.DMA,
        pltpu.SemaphoreType.DMA,
    ],
    compiler_params=_SC_PARAMS,
)


def _score_body(h, src2, dst2, out,
                sidx, didx, sra, dra, srb, drb, sv,
                gsa, gda, gsb, gdb):
    cid = lax.axis_index("c")
    sid = lax.axis_index("s")
    wid = cid * NS + sid

    pltpu.sync_copy(src2.at[wid], sidx)
    pltpu.sync_copy(dst2.at[wid], didx)

    def gathers(j, srows, drows, gs, gd):
        pltpu.async_copy(h.at[sidx.at[pl.ds(j * KS, KS)]], srows, gs)
        pltpu.async_copy(h.at[didx.at[pl.ds(j * KS, KS)]], drows, gd)

    def gathers_wait(srows, drows, gs, gd):
        pltpu.make_async_copy(h.at[sidx.at[pl.ds(0, KS)]], srows, gs).wait()
        pltpu.make_async_copy(h.at[didx.at[pl.ds(0, KS)]], drows, gd).wait()

    z16 = jnp.zeros((16,), jnp.float32)
    zid16 = jnp.zeros((16,), jnp.int32)

    def compute(j, srows, drows):
        base = j * KS
        for g in range(KS // 16):
            sv[pl.ds(base + g * 16, 16)] = z16
        for r in range(KS):
            m = [srows[r, pl.ds(v * 16, 16)] * drows[r, pl.ds(v * 16, 16)]
                 for v in range(D // 16)]
            m = [m[0] + m[1], m[2] + m[3], m[4] + m[5], m[6] + m[7]]
            p = (m[0] + m[1]) + (m[2] + m[3])
            plsc.addupdate_scatter(sv, [zid16 + (base + r)], p)

    gathers(0, sra, dra, gsa, gda)

    def blk(i, carry):
        a = 2 * i
        b = a + 1
        gathers_wait(sra, dra, gsa, gda)
        gathers(b, srb, drb, gsb, gdb)
        compute(a, sra, dra)
        gathers_wait(srb, drb, gsb, gdb)
        gathers(jnp.minimum(a + 2, NBS - 1), sra, dra, gsa, gda)
        compute(b, srb, drb)
        return carry

    lax.fori_loop(0, NBS // 2, blk, 0)
    gathers_wait(sra, dra, gsa, gda)
    pltpu.sync_copy(sv, out.at[pl.ds(wid * SPW, SPW)])


_score = pl.kernel(
    _score_body,
    out_type=jax.ShapeDtypeStruct((E2,), jnp.float32),
    mesh=_MESH,
    scratch_types=[
        pltpu.VMEM((SPW,), jnp.int32),
        pltpu.VMEM((SPW,), jnp.int32),
        pltpu.VMEM((KS, D), jnp.float32),
        pltpu.VMEM((KS, D), jnp.float32),
        pltpu.VMEM((KS, D), jnp.float32),
        pltpu.VMEM((KS, D), jnp.float32),
        pltpu.VMEM((SPW,), jnp.float32),
        pltpu.SemaphoreType.DMA,
        pltpu.SemaphoreType.DMA,
        pltpu.SemaphoreType.DMA,
        pltpu.SemaphoreType.DMA,
    ],
    compiler_params=_SC_PARAMS,
)


def _combine_body(relu, x, p0, p1, dall, ws, wn, b, out):
    deg = jnp.maximum(jnp.sum(dall[...], axis=1)[:, None], 1.0)
    hn = (p0[...] + p1[...]) / deg
    r = (jnp.dot(x[...], ws[...], preferred_element_type=jnp.float32,
                 precision=lax.Precision.HIGHEST)
         + jnp.dot(hn, wn[...], preferred_element_type=jnp.float32,
                   precision=lax.Precision.HIGHEST)
         + b[...])
    if relu:
        r = jnp.maximum(r, 0.0)
    out[...] = r


_RB = 1000


def _combine(x, p0, p1, dall, ws, wn, b, relu):
    return pl.pallas_call(
        functools.partial(_combine_body, relu),
        grid=(N // _RB,),
        in_specs=[
            pl.BlockSpec((_RB, D), lambda i: (i, 0)),
            pl.BlockSpec((_RB, D), lambda i: (i, 0)),
            pl.BlockSpec((_RB, D), lambda i: (i, 0)),
            pl.BlockSpec((_RB, NW), lambda i: (i, 0)),
            pl.BlockSpec((D, D), lambda i: (0, 0)),
            pl.BlockSpec((D, D), lambda i: (0, 0)),
            pl.BlockSpec((1, D), lambda i: (0, 0)),
        ],
        out_specs=pl.BlockSpec((_RB, D), lambda i: (i, 0)),
        out_shape=jax.ShapeDtypeStruct((N, D), jnp.float32),
    )(x, p0, p1, dall, ws, wn, b)


def kernel(x, edge_index, neg_edge_index, W_self1, W_neigh1, b1,
           W_self2, W_neigh2, b2):
    src = edge_index[0]
    dst = edge_index[1]
    src3 = src.reshape(NW, NBE, KE)
    dst3 = dst.reshape(NW, NBE, KE)
    zrows = jnp.zeros((N, D), jnp.float32)

    agg1, deg = _agg(x, src3, dst3, zrows)
    dall = deg.reshape(NW, N).T
    h1 = _combine(x, agg1[0], agg1[1], dall,
                  W_self1, W_neigh1, b1.reshape(1, D), relu=True)
    agg2, _deg2 = _agg(h1, src3, dst3, zrows)
    h2 = _combine(h1, agg2[0], agg2[1], dall,
                  W_self2, W_neigh2, b2.reshape(1, D), relu=False)

    srcs2 = jnp.concatenate([src, neg_edge_index[0]]).reshape(NW, SPW)
    dsts2 = jnp.concatenate([dst, neg_edge_index[1]]).reshape(NW, SPW)
    scores = _score(h2, srcs2, dsts2)
    return (scores[:E, None], scores[E:, None])


# speedup vs baseline: 1.1386x; 1.1386x over previous
import functools

import jax
import jax.numpy as jnp
from jax import lax
from jax.experimental import pallas as pl
from jax.experimental.pallas import tpu as pltpu
from jax.experimental.pallas import tpu_sc as plsc

N = 10000
E = 320000
D = 128

NC = 2
NS = 16
NW = NC * NS

EPW = E // NW
KE = 80
NBE = EPW // KE
PHB = 64
ROWS_PT = 624
ROWS_TAIL = N - NS * ROWS_PT

E2 = 2 * E
SPW = E2 // NW
KS = 80
NBS = SPW // KS

_MESH = plsc.VectorSubcoreMesh(
    core_axis_name="c", subcore_axis_name="s", num_cores=NC, num_subcores=NS)
_SC_PARAMS = pltpu.CompilerParams(needs_layout_passes=False)


def _zero_acc(zrows, acc, sid):
    rbase = sid * ROWS_PT
    tbase = NS * ROWS_PT
    pltpu.sync_copy(zrows.at[pl.ds(rbase, ROWS_PT)], acc.at[pl.ds(rbase, ROWS_PT)])

    @pl.when(sid == 0)
    def _():
        pltpu.sync_copy(zrows.at[pl.ds(tbase, ROWS_TAIL)],
                        acc.at[pl.ds(tbase, ROWS_TAIL)])


def _copy_out(acc, out, cid, sid):
    rbase = sid * ROWS_PT
    tbase = NS * ROWS_PT
    pltpu.sync_copy(acc.at[pl.ds(rbase, ROWS_PT)],
                    out.at[cid].at[pl.ds(rbase, ROWS_PT)])

    @pl.when(sid == 0)
    def _():
        pltpu.sync_copy(acc.at[pl.ds(tbase, ROWS_TAIL)],
                        out.at[cid].at[pl.ds(tbase, ROWS_TAIL)])


def _agg_body(h, src3, dst3, zrows, agg_out, deg_out,
              sidx, didx, rows_a, rows_b, degv, acc, sga, sgb, ssa, ssb):
    cid = lax.axis_index("c")
    sid = lax.axis_index("s")
    wid = cid * NS + sid

    _zero_acc(zrows, acc, sid)

    z16 = jnp.zeros((16,), jnp.float32)

    def zblk(i, carry):
        degv[pl.ds(i * 16, 16)] = z16
        return carry

    lax.fori_loop(0, N // 16, zblk, 0)
    plsc.subcore_barrier()

    def gather(j, rows, sem):
        return pltpu.async_copy(h.at[sidx.at[j]], rows, sem)

    def gather_wait(rows, sem):
        pltpu.make_async_copy(h.at[sidx.at[0]], rows, sem).wait()

    def scat(j, rows, sem):
        return pltpu.async_copy(rows, acc.at[didx.at[j]], sem, add=True)

    def scat_wait(rows, sem):
        pltpu.make_async_copy(rows, acc.at[didx.at[0]], sem).wait()

    ones16 = jnp.ones((16,), jnp.float32)

    def degcount(j):
        for g in range(KE // 16):
            ids = didx[j, pl.ds(g * 16, 16)]
            plsc.addupdate_scatter(degv, [ids], ones16)

    for blk0, pnb in ((0, PHB), (PHB, NBE - PHB)):
        pltpu.sync_copy(src3.at[wid].at[pl.ds(blk0, pnb)],
                        sidx.at[pl.ds(0, pnb)])
        pltpu.sync_copy(dst3.at[wid].at[pl.ds(blk0, pnb)],
                        didx.at[pl.ds(0, pnb)])

        gather(0, rows_a, sga)

        def blk(i, carry):
            a = 2 * i
            b = a + 1
            gather_wait(rows_a, sga)
            gather(b, rows_b, sgb)
            scat(a, rows_a, ssa)
            degcount(a)
            gather_wait(rows_b, sgb)
            scat_wait(rows_a, ssa)
            gather(jnp.minimum(a + 2, pnb - 1), rows_a, sga)
            scat(b, rows_b, ssb)
            degcount(b)
            scat_wait(rows_b, ssb)
            return carry

        lax.fori_loop(0, pnb // 2, blk, 0)
        gather_wait(rows_a, sga)
        if pnb % 2:
            scat(pnb - 1, rows_a, ssa)
            degcount(pnb - 1)
            scat_wait(rows_a, ssa)

    plsc.subcore_barrier()
    _copy_out(acc, agg_out, cid, sid)
    pltpu.sync_copy(degv, deg_out.at[cid].at[sid])


_agg = pl.kernel(
    _agg_body,
    out_type=(jax.ShapeDtypeStruct((NC, N, D), jnp.float32),
              jax.ShapeDtypeStruct((NC, NS, N), jnp.float32)),
    mesh=_MESH,
    scratch_types=[
        pltpu.VMEM((PHB, KE), jnp.int32),
        pltpu.VMEM((PHB, KE), jnp.int32),
        pltpu.VMEM((KE, D), jnp.float32),
        pltpu.VMEM((KE, D), jnp.float32),
        pltpu.VMEM((N,), jnp.float32),
        pltpu.VMEM_SHARED((N, D), jnp.float32),
        pltpu.SemaphoreType.DMA,
        pltpu.SemaphoreType.DMA,
        pltpu.SemaphoreType.DMA,
        pltpu.SemaphoreType.DMA,
    ],
    compiler_params=_SC_PARAMS,
)


def _score_body(h, src2, dst2, out,
                sidx, didx, sra, dra, srb, drb, sv,
                gsa, gda, gsb, gdb):
    cid = lax.axis_index("c")
    sid = lax.axis_index("s")
    wid = cid * NS + sid

    pltpu.sync_copy(src2.at[wid], sidx)
    pltpu.sync_copy(dst2.at[wid], didx)

    def gathers(j, srows, drows, gs, gd):
        pltpu.async_copy(h.at[sidx.at[pl.ds(j * KS, KS)]], srows, gs)
        pltpu.async_copy(h.at[didx.at[pl.ds(j * KS, KS)]], drows, gd)

    def gathers_wait(srows, drows, gs, gd):
        pltpu.make_async_copy(h.at[sidx.at[pl.ds(0, KS)]], srows, gs).wait()
        pltpu.make_async_copy(h.at[didx.at[pl.ds(0, KS)]], drows, gd).wait()

    lane = lax.iota(jnp.int32, 16)

    def compute(j, srows, drows):
        base = j * KS
        for g in range(KS // 16):
            score = jnp.zeros((16,), jnp.float32)
            for e in range(16):
                r = g * 16 + e
                m = [srows[r, pl.ds(v * 16, 16)] * drows[r, pl.ds(v * 16, 16)]
                     for v in range(D // 16)]
                m = [m[0] + m[1], m[2] + m[3], m[4] + m[5], m[6] + m[7]]
                p = (m[0] + m[1]) + (m[2] + m[3])
                score = jnp.where(lane == e, jnp.sum(p), score)
            sv[pl.ds(base + g * 16, 16)] = score

    gathers(0, sra, dra, gsa, gda)

    def blk(i, carry):
        a = 2 * i
        b = a + 1
        gathers_wait(sra, dra, gsa, gda)
        gathers(b, srb, drb, gsb, gdb)
        compute(a, sra, dra)
        gathers_wait(srb, drb, gsb, gdb)
        gathers(jnp.minimum(a + 2, NBS - 1), sra, dra, gsa, gda)
        compute(b, srb, drb)
        return carry

    lax.fori_loop(0, NBS // 2, blk, 0)
    gathers_wait(sra, dra, gsa, gda)
    pltpu.sync_copy(sv, out.at[pl.ds(wid * SPW, SPW)])


_score = pl.kernel(
    _score_body,
    out_type=jax.ShapeDtypeStruct((E2,), jnp.float32),
    mesh=_MESH,
    scratch_types=[
        pltpu.VMEM((SPW,), jnp.int32),
        pltpu.VMEM((SPW,), jnp.int32),
        pltpu.VMEM((KS, D), jnp.float32),
        pltpu.VMEM((KS, D), jnp.float32),
        pltpu.VMEM((KS, D), jnp.float32),
        pltpu.VMEM((KS, D), jnp.float32),
        pltpu.VMEM((SPW,), jnp.float32),
        pltpu.SemaphoreType.DMA,
        pltpu.SemaphoreType.DMA,
        pltpu.SemaphoreType.DMA,
        pltpu.SemaphoreType.DMA,
    ],
    compiler_params=_SC_PARAMS,
)


def _combine_body(relu, x, p0, p1, dall, ws, wn, b, out):
    deg = jnp.maximum(jnp.sum(dall[...], axis=1)[:, None], 1.0)
    hn = (p0[...] + p1[...]) / deg
    r = (jnp.dot(x[...], ws[...], preferred_element_type=jnp.float32,
                 precision=lax.Precision.HIGHEST)
         + jnp.dot(hn, wn[...], preferred_element_type=jnp.float32,
                   precision=lax.Precision.HIGHEST)
         + b[...])
    if relu:
        r = jnp.maximum(r, 0.0)
    out[...] = r


_RB = 1000


def _combine(x, p0, p1, dall, ws, wn, b, relu):
    return pl.pallas_call(
        functools.partial(_combine_body, relu),
        grid=(N // _RB,),
        in_specs=[
            pl.BlockSpec((_RB, D), lambda i: (i, 0)),
            pl.BlockSpec((_RB, D), lambda i: (i, 0)),
            pl.BlockSpec((_RB, D), lambda i: (i, 0)),
            pl.BlockSpec((_RB, NW), lambda i: (i, 0)),
            pl.BlockSpec((D, D), lambda i: (0, 0)),
            pl.BlockSpec((D, D), lambda i: (0, 0)),
            pl.BlockSpec((1, D), lambda i: (0, 0)),
        ],
        out_specs=pl.BlockSpec((_RB, D), lambda i: (i, 0)),
        out_shape=jax.ShapeDtypeStruct((N, D), jnp.float32),
    )(x, p0, p1, dall, ws, wn, b)


def kernel(x, edge_index, neg_edge_index, W_self1, W_neigh1, b1,
           W_self2, W_neigh2, b2):
    src = edge_index[0]
    dst = edge_index[1]
    src3 = src.reshape(NW, NBE, KE)
    dst3 = dst.reshape(NW, NBE, KE)
    zrows = jnp.zeros((N, D), jnp.float32)

    agg1, deg = _agg(x, src3, dst3, zrows)
    dall = deg.reshape(NW, N).T
    h1 = _combine(x, agg1[0], agg1[1], dall,
                  W_self1, W_neigh1, b1.reshape(1, D), relu=True)
    agg2, _deg2 = _agg(h1, src3, dst3, zrows)
    h2 = _combine(h1, agg2[0], agg2[1], dall,
                  W_self2, W_neigh2, b2.reshape(1, D), relu=False)

    srcs2 = jnp.concatenate([src, neg_edge_index[0]]).reshape(NW, SPW)
    dsts2 = jnp.concatenate([dst, neg_edge_index[1]]).reshape(NW, SPW)
    scores = _score(h2, srcs2, dsts2)
    return (scores[:E, None], scores[E:, None])
